# Initial kernel scaffold; baseline (speedup 1.0000x reference)
import jax
import jax.numpy as jnp
from jax.experimental import pallas as pl


def kernel(bias_table, sequence_length):
    raise NotImplementedError("write your pallas kernel here")


# speedup vs baseline: 40.5488x; 40.5488x over previous
import math

import numpy as np
import jax
import jax.numpy as jnp
from jax import lax
from jax.experimental import pallas as pl
from jax.experimental.pallas import tpu as pltpu
from jax.experimental.pallas import tpu_sc as plsc

_H = 16
_NB = 32
_S = 2048
_M = 2 * _S - 1
_W = 4112
_BP = 4128
_RB = 16
_NBLK = (_S // 2) // _RB


def _bucket_ids_ext() -> np.ndarray:
    q = np.arange(_M)
    d = (_S - 1) - q
    dist = np.maximum(d, 0)
    small = dist < (_NB // 2)
    ratio = np.log(np.maximum(dist.astype(np.float64), 1.0) / 16.0) / math.log(8.0)
    large = np.minimum(16 + (ratio * 16.0).astype(np.int64), _NB - 1)
    ids = np.where(small, dist, large).astype(np.int32)
    return np.concatenate([
        np.full(8, ids[0], np.int32),
        ids,
        np.full(_BP - 8 - _M, ids[-1], np.int32),
    ])


_BUCKET_EXT = _bucket_ids_ext()


def _sc_body(tab_hbm, bidx_hbm, out_hbm, bidx_v, tab_v, gr_v, a_v, sem):
    cid = lax.axis_index("c")
    sid = lax.axis_index("s")
    wid = sid * 2 + cid
    h = wid // 2
    half = wid % 2

    pltpu.sync_copy(bidx_hbm, bidx_v)
    pltpu.sync_copy(tab_hbm, tab_v)

    hvec = jnp.full((16,), 0, jnp.int32) + h

    def gather_body(i, carry):
        b16 = bidx_v[pl.ds(i * 16, 16)]
        gr_v[pl.ds(i * 16, 16)] = plsc.load_gather(tab_v, [hvec, b16])
        return carry

    lax.fori_loop(0, _BP // 16, gather_body, 0)

    for r in range(_RB):
        off = 15 - r

        def shift_body(ci, carry, _off=off, _r=r):
            a_v[_r, pl.ds(ci * 16, 16)] = gr_v[pl.ds(ci * 16 + _off, 16)]
            return carry

        lax.fori_loop(0, _W // 16, shift_body, 0)

    base = half * (_S // 2)
    copies = []
    for b in range(_NBLK):
        i0 = base + _RB * b
        s0 = pl.multiple_of(2040 - i0, 8)
        cp = pltpu.make_async_copy(
            a_v.at[:, pl.ds(s0, _S)],
            out_hbm.at[h, pl.ds(i0, _RB)],
            sem,
        )
        cp.start()
        copies.append(cp)
    for cp in copies:
        cp.wait()


def kernel(bias_table, sequence_length):
    del sequence_length
    tab_t = bias_table.T
    bidx = jnp.asarray(_BUCKET_EXT)
    mesh = plsc.VectorSubcoreMesh(
        core_axis_name="c", subcore_axis_name="s", num_cores=2, num_subcores=16
    )
    run = pl.kernel(
        _sc_body,
        out_type=jax.ShapeDtypeStruct((_H, _S, _S), jnp.float32),
        mesh=mesh,
        scratch_types=[
            pltpu.VMEM((_BP,), jnp.int32),
            pltpu.VMEM((_H, _NB), jnp.float32),
            pltpu.VMEM((_BP,), jnp.float32),
            pltpu.VMEM((_RB, _W), jnp.float32),
            pltpu.SemaphoreType.DMA,
        ],
        compiler_params=pltpu.CompilerParams(
            use_tc_tiling_on_sc=False, needs_layout_passes=False
        ),
    )
    return run(tab_t, bidx)


# speedup vs baseline: 96.3519x; 2.3762x over previous
import math

import numpy as np
import jax
import jax.numpy as jnp
from jax import lax
from jax.experimental import pallas as pl
from jax.experimental.pallas import tpu as pltpu
from jax.experimental.pallas import tpu_sc as plsc

_H = 16
_NB = 32
_S = 2048
_M = 2 * _S - 1
_GP = 4096
_VW = 3968
_NS = 16
_HPC = 8


def _bucket_ids() -> np.ndarray:
    q = np.arange(_M)
    d = (_S - 1) - q
    dist = np.maximum(d, 0)
    small = dist < (_NB // 2)
    ratio = np.log(np.maximum(dist.astype(np.float64), 1.0) / 16.0) / math.log(8.0)
    large = np.minimum(16 + (ratio * 16.0).astype(np.int64), _NB - 1)
    ids = np.where(small, dist, large).astype(np.int32)
    return np.concatenate([ids, np.zeros(_GP - _M, np.int32)])


_BUCKET_IDS = _bucket_ids()


def _sc_body(tab_hbm, bidx_hbm, out_hbm, bidx_v, tab_v, gr_v, tmp_v, v_sh,
             sem_stream):
    cid = lax.axis_index("c")
    tid = lax.axis_index("s")

    pltpu.sync_copy(bidx_hbm, bidx_v)
    pltpu.sync_copy(tab_hbm, tab_v)

    row0 = pl.multiple_of(8 * tid, 8)
    s_t = pl.multiple_of(1920 - 128 * tid, 128)
    i_t = pl.multiple_of(128 * tid, 8)

    streams = []
    for w in range(_HPC):
        h = cid * _HPC + w
        b = w % 2

        hoff = h * _NB

        def gather_body(i, carry):
            b16 = bidx_v[pl.ds(i * 16, 16)]
            gr_v[pl.ds(i * 16, 16)] = plsc.load_gather(tab_v, [hoff + b16])
            return carry

        lax.fori_loop(0, _GP // 16, gather_body, 0)

        for r in range(8):
            off = (127 - r) - 8 * tid

            def shift_body(ci, carry, _r=r, _off=off):
                tmp_v[_r, pl.ds(ci * 16, 16)] = gr_v[pl.ds(ci * 16 + _off, 16)]
                return carry

            lax.fori_loop(0, _VW // 16, shift_body, 0)

        if w >= 2:
            streams[w - 2].wait()
        plsc.subcore_barrier()
        pltpu.sync_copy(tmp_v, v_sh.at[b, pl.ds(row0, 8), :])
        plsc.subcore_barrier()
        cp = pltpu.make_async_copy(
            v_sh.at[b, :, pl.ds(s_t, _S)],
            out_hbm.at[h, pl.ds(i_t, 128)],
            sem_stream,
        )
        cp.start()
        streams.append(cp)

    streams[_HPC - 2].wait()
    streams[_HPC - 1].wait()


def kernel(bias_table, sequence_length):
    del sequence_length
    tab_flat = bias_table.T.reshape(-1)
    bidx = jnp.asarray(_BUCKET_IDS)
    mesh = plsc.VectorSubcoreMesh(
        core_axis_name="c", subcore_axis_name="s", num_cores=2, num_subcores=_NS
    )
    run = pl.kernel(
        _sc_body,
        out_type=jax.ShapeDtypeStruct((_H, _S, _S), jnp.float32),
        mesh=mesh,
        scratch_types=[
            pltpu.VMEM((_GP,), jnp.int32),
            pltpu.VMEM((_H * _NB,), jnp.float32),
            pltpu.VMEM((_GP,), jnp.float32),
            pltpu.VMEM((8, _VW), jnp.float32),
            pltpu.VMEM_SHARED((2, 128, _VW), jnp.float32),
            pltpu.SemaphoreType.DMA,
        ],
        compiler_params=pltpu.CompilerParams(needs_layout_passes=False),
    )
    return run(tab_flat, bidx)
